# baseline (speedup 1.0000x reference)
import functools

import jax
import jax.numpy as jnp
from jax import lax
from jax.experimental import pallas as pl
from jax.experimental.pallas import tpu as pltpu
from jax.experimental.pallas import tpu_sc as plsc

R = 1024
N = 1024 * 1024
NC, NS = 2, 16
NW = NC * NS
NPW = N // NW
CH = 512
NCHUNK = NPW // CH
NPAIR = NCHUNK // 2
NG = CH // 16
NJ2 = (2 * CH) // 128
JMAX = 6 * R * R * 3 // 8 - 1

NUNIT = 6 * 128
UPW = NUNIT // NW
TROW = 8 * R


def _cf(v):
    return jnp.full((16,), v, dtype=jnp.float32)


def _ci(v):
    return jnp.full((16,), v, dtype=jnp.int32)


def _fmt_body(src_hbm, dst_hbm, c0a, c1a, c2a, c0b, c1b, c2b, oba, obb,
              semia, semib, semoa, semob):
    wid = lax.axis_index("s") * NC + lax.axis_index("c")
    iota = lax.iota(jnp.int32, 16)
    iota3 = iota * 3

    def in_copies(uu, bufs, sem):
        unit = wid * UPW + uu
        f = unit // 128
        tv = unit % 128
        base_in = (f * 3 * 128 + tv) * TROW
        return [pltpu.make_async_copy(
                    src_hbm.at[pl.ds(base_in + k * 128 * TROW, TROW)],
                    bufs[k], sem) for k in range(3)]

    def out_copy(uu, ob, sem):
        unit = wid * UPW + uu
        f = unit // 128
        tv = unit % 128
        return pltpu.make_async_copy(
            ob, dst_hbm.at[pl.ds((f * 1024 * 1024 + tv * TROW) * 3,
                                 3 * TROW)], sem)

    for cp in in_copies(0, (c0a, c1a, c2a), semia):
        cp.start()

    def interleave(bufs, ob):
        c0, c1, c2 = bufs

        def grp(g, carry2):
            vs = g >> 6
            tu = (g >> 3) & 7
            j = g & 7
            s_in = tu * 1024 + vs * 128 + j * 16
            s_out = (vs * 1024 + tu * 128 + j * 16) * 3
            sl = pl.ds(s_in, 16)
            ix = iota3 + s_out
            plsc.store_scatter(ob, [ix], c0[sl])
            plsc.store_scatter(ob, [ix + 1], c1[sl])
            plsc.store_scatter(ob, [ix + 2], c2[sl])
            return carry2

        lax.fori_loop(0, 512, grp, 0, unroll=8)

    def unit_body(uu, carry):
        bufs = [(c0a, c1a, c2a), (c0b, c1b, c2b)]
        obs = [oba, obb]
        semis = [semia, semib]
        semos = [semoa, semob]
        for p in range(2):
            u = uu * 2 + p
            for cp in in_copies(u, bufs[p], semis[p]):
                cp.wait()

            @pl.when(u + 1 < UPW)
            def _():
                for cp in in_copies(u + 1, bufs[1 - p], semis[1 - p]):
                    cp.start()

            @pl.when(u >= 2)
            def _():
                out_copy(u - 2, obs[p], semos[p]).wait()

            interleave(bufs[p], obs[p])
            out_copy(u, obs[p], semos[p]).start()
        return carry

    lax.fori_loop(0, UPW // 2, unit_body, 0)
    out_copy(UPW - 2, oba, semoa).wait()
    out_copy(UPW - 1, obb, semob).wait()


def _calc_group_math(x, y, z):
    X, Y, Z = x, z, -y
    ax, ay, az = jnp.abs(X), jnp.abs(Y), jnp.abs(Z)
    is_x = (ax >= ay) & (ax >= az)
    is_y = (~is_x) & (ay >= az)
    xpos, ypos, zpos = X > _cf(0.0), Y > _cf(0.0), Z > _cf(0.0)
    one = _ci(1)
    face = jnp.where(is_x, jnp.where(xpos, one * 0, one),
                     jnp.where(is_y, jnp.where(ypos, one * 2, one * 3),
                               jnp.where(zpos, one * 4, one * 5)))
    ma = jnp.maximum(jnp.maximum(jnp.maximum(ax, ay), az), _cf(1e-20))
    rcp = _cf(1.0) / ma
    sc = jnp.where(is_x, jnp.where(xpos, -Z, Z),
                   jnp.where(is_y, X, jnp.where(zpos, X, -X)))
    tc = jnp.where(is_x, -Y,
                   jnp.where(is_y, jnp.where(ypos, Z, -Z), -Y))
    u = (_cf(0.5) * (sc * rcp + _cf(1.0))) * _cf(float(R)) - _cf(0.5)
    v = (_cf(0.5) * (tc * rcp + _cf(1.0))) * _cf(float(R)) - _cf(0.5)
    u0t = (u + _cf(1.0)).astype(jnp.int32) - one
    v0t = (v + _cf(1.0)).astype(jnp.int32) - one
    fu = u - u0t.astype(jnp.float32)
    fv = v - v0t.astype(jnp.float32)
    zero = one * 0
    rmax = _ci(R - 1)
    u0i = jnp.maximum(jnp.minimum(u0t, rmax), zero)
    v0i = jnp.maximum(jnp.minimum(v0t, rmax), zero)
    v1i = jnp.minimum(v0i + one, rmax)
    fidx = face << 20
    t0 = fidx + (v0i << 10) + u0i
    t1 = fidx + (v1i << 10) + u0i
    w0 = (t0 << 1) + t0
    w1 = (t1 << 1) + t1
    j00 = w0 >> 3
    j10 = w1 >> 3
    j01 = jnp.minimum(j00 + one, _ci(JMAX))
    j11 = jnp.minimum(j10 + one, _ci(JMAX))
    du = jnp.where(u0i < rmax, one, zero)
    pk = (w0 & _ci(7)) | ((w1 & _ci(7)) << 3) | (du << 6)
    return j00, j01, j10, j11, pk, fu, fv


def _sc_body(l_hbm, tab_hbm, out_hbm, *refs):
    sets = [dict(zip(("dx", "dy", "dz", "ib0", "ib1", "pk", "fu", "fv",
                      "rb0", "rb1", "ox", "oy", "oz",
                      "semd", "semg", "semo"), refs[i * 16:(i + 1) * 16]))
            for i in range(4)]
    wid = lax.axis_index("s") * NC + lax.axis_index("c")
    iota = lax.iota(jnp.int32, 16)
    iota2 = iota * 2

    def dir_copies(c, bufs, sem):
        p0 = wid * NPW + c * CH
        return [pltpu.make_async_copy(
                    l_hbm.at[pl.ds(p0 + k * N, CH)], bufs[k], sem)
                for k in range(3)]

    def out_copies(c, bufs, sem):
        p0 = wid * NPW + c * CH
        return [pltpu.make_async_copy(
                    bufs[k], out_hbm.at[pl.ds(p0 + k * N, CH)], sem)
                for k in range(3)]

    def gather_copies(ib0, ib1, rb0, rb1, sem):
        return [pltpu.make_async_copy(tab_hbm.at[ib0], rb0, sem),
                pltpu.make_async_copy(tab_hbm.at[ib1], rb1, sem)]

    def calc_chunk(dirs, ib0, ib1, pkb_, fub_, fvb_):
        dx, dy, dz = dirs

        def calc_group(g, carry2):
            sl = pl.ds(g * 16, 16)
            j00, j01, j10, j11, pk, fu, fv = _calc_group_math(
                dx[sl], dy[sl], dz[sl])
            base2 = iota2 + g * 32
            plsc.store_scatter(ib0, [base2], j00)
            plsc.store_scatter(ib0, [base2 + 1], j01)
            plsc.store_scatter(ib1, [base2], j10)
            plsc.store_scatter(ib1, [base2 + 1], j11)
            pkb_[sl] = pk
            fub_[sl] = fu
            fvb_[sl] = fv
            return carry2

        lax.fori_loop(0, NG, calc_group, 0, unroll=4)

    def blend_chunk(rb0, rb1, pkb_, fub_, fvb_, outs):
        ox, oy, oz = outs

        def blend_group(g, carry2):
            sl = pl.ds(g * 16, 16)
            fu = fub_[sl]
            fv = fvb_[sl]
            pk = pkb_[sl]
            seven = _ci(7)
            o0 = pk & seven
            o1 = (pk >> 3) & seven
            du3 = (pk >> 6) * 3
            r16 = (iota + g * 16) << 4
            q00 = r16 + o0
            q01 = q00 + du3
            q10 = r16 + o1
            q11 = q10 + du3
            res = []
            for k in range(3):
                kk = _ci(k)
                a = q00 + kk
                c00 = plsc.load_gather(rb0, [a >> 3, a & seven])
                a = q01 + kk
                c01 = plsc.load_gather(rb0, [a >> 3, a & seven])
                a = q10 + kk
                c10 = plsc.load_gather(rb1, [a >> 3, a & seven])
                a = q11 + kk
                c11 = plsc.load_gather(rb1, [a >> 3, a & seven])
                top = c00 + fu * (c01 - c00)
                bot = c10 + fu * (c11 - c10)
                res.append(top + fv * (bot - top))
            ox[sl] = res[0]
            oy[sl] = res[1]
            oz[sl] = res[2]
            return carry2

        lax.fori_loop(0, NG, blend_group, 0, unroll=4)

    def dirs_of(s):
        return (s["dx"], s["dy"], s["dz"])

    def outs_of(s):
        return (s["ox"], s["oy"], s["oz"])

    def fire_dirs(c, s):
        for cp in dir_copies(c, dirs_of(s), s["semd"]):
            cp.start()

    def calc_fire(c, s):
        for cp in dir_copies(c, dirs_of(s), s["semd"]):
            cp.wait()
        calc_chunk(dirs_of(s), s["ib0"], s["ib1"], s["pk"], s["fu"], s["fv"])
        for cp in gather_copies(s["ib0"], s["ib1"], s["rb0"], s["rb1"],
                                s["semg"]):
            cp.start()

    def drain_blend(c, s, first):
        @pl.when(jnp.logical_not(first))
        def _():
            for cp in out_copies(c - 4, outs_of(s), s["semo"]):
                cp.wait()
        for cp in gather_copies(s["ib0"], s["ib1"], s["rb0"], s["rb1"],
                                s["semg"]):
            cp.wait()
        blend_chunk(s["rb0"], s["rb1"], s["pk"], s["fu"], s["fv"], outs_of(s))
        for cp in out_copies(c, outs_of(s), s["semo"]):
            cp.start()

    A, B, Cs, D = sets
    fire_dirs(0, A)
    fire_dirs(1, B)
    fire_dirs(2, Cs)
    fire_dirs(3, D)
    calc_fire(0, A)
    calc_fire(1, B)

    def quad_body(j, carry):
        c = j * 4
        calc_fire(c + 2, Cs)
        calc_fire(c + 3, D)

        @pl.when(c + 4 < NCHUNK)
        def _():
            fire_dirs(c + 4, A)
            fire_dirs(c + 5, B)
        drain_blend(c, A, j == 0)
        drain_blend(c + 1, B, j == 0)

        @pl.when(c + 4 < NCHUNK)
        def _():
            calc_fire(c + 4, A)
            calc_fire(c + 5, B)

        @pl.when(c + 6 < NCHUNK)
        def _():
            fire_dirs(c + 6, Cs)
            fire_dirs(c + 7, D)
        drain_blend(c + 2, Cs, j == 0)
        drain_blend(c + 3, D, j == 0)
        return carry

    lax.fori_loop(0, NCHUNK // 4, quad_body, 0)
    for cp in out_copies(NCHUNK - 4, outs_of(A), A["semo"]):
        cp.wait()
    for cp in out_copies(NCHUNK - 3, outs_of(B), B["semo"]):
        cp.wait()
    for cp in out_copies(NCHUNK - 2, outs_of(Cs), Cs["semo"]):
        cp.wait()
    for cp in out_copies(NCHUNK - 1, outs_of(D), D["semo"]):
        cp.wait()


@jax.jit
def _run(lf, tbf):
    mesh = plsc.VectorSubcoreMesh(core_axis_name="c", subcore_axis_name="s")
    cp = pltpu.CompilerParams(needs_layout_passes=False,
                              use_tc_tiling_on_sc=False)
    fmt = pl.kernel(
        _fmt_body,
        out_type=jax.ShapeDtypeStruct((6 * R * R * 3,), jnp.float32),
        mesh=mesh,
        compiler_params=cp,
        scratch_types=(
            [pltpu.VMEM((TROW,), jnp.float32)] * 6
            + [pltpu.VMEM((3 * TROW,), jnp.float32)] * 2
            + [pltpu.SemaphoreType.DMA] * 4
        ),
    )
    tab = fmt(tbf).reshape(-1, 8)
    f = pl.kernel(
        _sc_body,
        out_type=jax.ShapeDtypeStruct((N * 3,), jnp.float32),
        mesh=mesh,
        compiler_params=cp,
        scratch_types=(
            [
                t for _ in range(4) for t in (
                    [pltpu.VMEM((CH,), jnp.float32)] * 3
                    + [pltpu.VMEM((2 * CH,), jnp.int32)] * 2
                    + [pltpu.VMEM((CH,), jnp.int32)]
                    + [pltpu.VMEM((CH,), jnp.float32)] * 2
                    + [pltpu.VMEM((2 * CH, 8), jnp.float32)] * 2
                    + [pltpu.VMEM((CH,), jnp.float32)] * 3
                    + [pltpu.SemaphoreType.DMA] * 3
                )
            ]
        ),
    )
    return f(lf, tab)


def kernel(l, base):
    lf = (l.transpose(2, 0, 1).reshape(3, 128, 8, 8, 128)
           .transpose(0, 1, 3, 2, 4).reshape(-1))
    tbf = (base.transpose(0, 3, 1, 2).reshape(6, 3, 128, 8, 8, 128)
               .transpose(0, 1, 2, 4, 3, 5).reshape(-1))
    out = _run(lf, tbf)
    return (out.reshape(3, 128, 8, 8, 128).transpose(0, 1, 3, 2, 4)
               .reshape(3, 1024, 1024).transpose(1, 2, 0))


# speedup vs baseline: 1.0037x; 1.0037x over previous
import jax
import jax.numpy as jnp
from jax import lax
from jax.experimental import pallas as pl
from jax.experimental.pallas import tpu as pltpu
from jax.experimental.pallas import tpu_sc as plsc

R = 1024
N = 1024 * 1024
NC, NS = 2, 16
NW = NC * NS
NPW = N // NW
CH = 512
NCHUNK = NPW // CH
NPAIR = NCHUNK // 2
NG = CH // 16
NJ2 = (2 * CH) // 128
JMAX = 6 * R * R * 3 // 8 - 1

NUNIT = 6 * 128
UPW = NUNIT // NW
TROW = 8 * R


def _cf(v):
    return jnp.full((16,), v, dtype=jnp.float32)


def _ci(v):
    return jnp.full((16,), v, dtype=jnp.int32)


def _fmt_body(src_hbm, dst_hbm, c0a, c1a, c2a, c0b, c1b, c2b, oba, obb,
              semia, semib, semoa, semob):
    wid = lax.axis_index("s") * NC + lax.axis_index("c")
    iota = lax.iota(jnp.int32, 16)
    iota3 = iota * 3

    def in_copies(uu, bufs, sem):
        unit = wid * UPW + uu
        f = unit // 128
        tv = unit % 128
        base_in = (f * 3 * 128 + tv) * TROW
        return [pltpu.make_async_copy(
                    src_hbm.at[pl.ds(base_in + k * 128 * TROW, TROW)],
                    bufs[k], sem) for k in range(3)]

    def out_copy(uu, ob, sem):
        unit = wid * UPW + uu
        f = unit // 128
        tv = unit % 128
        return pltpu.make_async_copy(
            ob, dst_hbm.at[pl.ds((f * 1024 * 1024 + tv * TROW) * 3,
                                 3 * TROW)], sem)

    for cp in in_copies(0, (c0a, c1a, c2a), semia):
        cp.start()

    def interleave(bufs, ob):
        c0, c1, c2 = bufs

        def grp(g, carry2):
            vs = g >> 6
            tu = (g >> 3) & 7
            j = g & 7
            s_in = tu * 1024 + vs * 128 + j * 16
            s_out = (vs * 1024 + tu * 128 + j * 16) * 3
            sl = pl.ds(s_in, 16)
            ix = iota3 + s_out
            plsc.store_scatter(ob, [ix], c0[sl])
            plsc.store_scatter(ob, [ix + 1], c1[sl])
            plsc.store_scatter(ob, [ix + 2], c2[sl])
            return carry2

        lax.fori_loop(0, 512, grp, 0, unroll=8)

    def unit_body(uu, carry):
        bufs = [(c0a, c1a, c2a), (c0b, c1b, c2b)]
        obs = [oba, obb]
        semis = [semia, semib]
        semos = [semoa, semob]
        for p in range(2):
            u = uu * 2 + p
            for cp in in_copies(u, bufs[p], semis[p]):
                cp.wait()

            @pl.when(u + 1 < UPW)
            def _():
                for cp in in_copies(u + 1, bufs[1 - p], semis[1 - p]):
                    cp.start()

            @pl.when(u >= 2)
            def _():
                out_copy(u - 2, obs[p], semos[p]).wait()

            interleave(bufs[p], obs[p])
            out_copy(u, obs[p], semos[p]).start()
        return carry

    lax.fori_loop(0, UPW // 2, unit_body, 0)
    out_copy(UPW - 2, oba, semoa).wait()
    out_copy(UPW - 1, obb, semob).wait()


def _calc_group_math(x, y, z):
    X, Y, Z = x, z, -y
    ax, ay, az = jnp.abs(X), jnp.abs(Y), jnp.abs(Z)
    is_x = (ax >= ay) & (ax >= az)
    is_y = (~is_x) & (ay >= az)
    xpos, ypos, zpos = X > _cf(0.0), Y > _cf(0.0), Z > _cf(0.0)
    one = _ci(1)
    face = jnp.where(is_x, jnp.where(xpos, one * 0, one),
                     jnp.where(is_y, jnp.where(ypos, one * 2, one * 3),
                               jnp.where(zpos, one * 4, one * 5)))
    ma = jnp.maximum(jnp.maximum(jnp.maximum(ax, ay), az), _cf(1e-20))
    rcp = _cf(1.0) / ma
    sc = jnp.where(is_x, jnp.where(xpos, -Z, Z),
                   jnp.where(is_y, X, jnp.where(zpos, X, -X)))
    tc = jnp.where(is_x, -Y,
                   jnp.where(is_y, jnp.where(ypos, Z, -Z), -Y))
    u = (_cf(0.5) * (sc * rcp + _cf(1.0))) * _cf(float(R)) - _cf(0.5)
    v = (_cf(0.5) * (tc * rcp + _cf(1.0))) * _cf(float(R)) - _cf(0.5)
    u0t = (u + _cf(1.0)).astype(jnp.int32) - one
    v0t = (v + _cf(1.0)).astype(jnp.int32) - one
    fu = u - u0t.astype(jnp.float32)
    fv = v - v0t.astype(jnp.float32)
    zero = one * 0
    rmax = _ci(R - 1)
    u0i = jnp.maximum(jnp.minimum(u0t, rmax), zero)
    v0i = jnp.maximum(jnp.minimum(v0t, rmax), zero)
    v1i = jnp.minimum(v0i + one, rmax)
    fidx = face << 20
    t0 = fidx + (v0i << 10) + u0i
    t1 = fidx + (v1i << 10) + u0i
    w0 = (t0 << 1) + t0
    w1 = (t1 << 1) + t1
    j00 = w0 >> 3
    j10 = w1 >> 3
    j01 = jnp.minimum(j00 + one, _ci(JMAX))
    j11 = jnp.minimum(j10 + one, _ci(JMAX))
    du = jnp.where(u0i < rmax, one, zero)
    pk = (w0 & _ci(7)) | ((w1 & _ci(7)) << 3) | (du << 6)
    return j00, j01, j10, j11, pk, fu, fv


def _sc_body(l_hbm, tab_hbm, out_hbm, *refs):
    sets = [dict(zip(("dx", "dy", "dz", "ib0", "ib1", "pk", "fu", "fv",
                      "rb0", "rb1", "ox", "oy", "oz",
                      "semd", "semg", "semo"), refs[i * 16:(i + 1) * 16]))
            for i in range(4)]
    wid = lax.axis_index("s") * NC + lax.axis_index("c")
    iota = lax.iota(jnp.int32, 16)
    iota2 = iota * 2

    def dir_copies(c, bufs, sem):
        p0 = wid * NPW + c * CH
        return [pltpu.make_async_copy(
                    l_hbm.at[pl.ds(p0 + k * N, CH)], bufs[k], sem)
                for k in range(3)]

    def out_copies(c, bufs, sem):
        p0 = wid * NPW + c * CH
        return [pltpu.make_async_copy(
                    bufs[k], out_hbm.at[pl.ds(p0 + k * N, CH)], sem)
                for k in range(3)]

    def gather_copies(ib0, ib1, rb0, rb1, sem):
        return [pltpu.make_async_copy(tab_hbm.at[ib0], rb0, sem),
                pltpu.make_async_copy(tab_hbm.at[ib1], rb1, sem)]

    def calc_chunk(dirs, ib0, ib1, pkb_, fub_, fvb_):
        dx, dy, dz = dirs

        def calc_group(g, carry2):
            sl = pl.ds(g * 16, 16)
            j00, j01, j10, j11, pk, fu, fv = _calc_group_math(
                dx[sl], dy[sl], dz[sl])
            base2 = iota2 + g * 32
            plsc.store_scatter(ib0, [base2], j00)
            plsc.store_scatter(ib0, [base2 + 1], j01)
            plsc.store_scatter(ib1, [base2], j10)
            plsc.store_scatter(ib1, [base2 + 1], j11)
            pkb_[sl] = pk
            fub_[sl] = fu
            fvb_[sl] = fv
            return carry2

        lax.fori_loop(0, NG, calc_group, 0, unroll=4)

    def blend_chunk(rb0, rb1, pkb_, fub_, fvb_, outs):
        ox, oy, oz = outs

        def blend_group(g, carry2):
            sl = pl.ds(g * 16, 16)
            fu = fub_[sl]
            fv = fvb_[sl]
            pk = pkb_[sl]
            seven = _ci(7)
            o0 = pk & seven
            o1 = (pk >> 3) & seven
            du3 = (pk >> 6) * 3
            r16 = (iota + g * 16) << 4
            q00 = r16 + o0
            q01 = q00 + du3
            q10 = r16 + o1
            q11 = q10 + du3
            res = []
            for k in range(3):
                kk = _ci(k)
                a = q00 + kk
                c00 = plsc.load_gather(rb0, [a >> 3, a & seven])
                a = q01 + kk
                c01 = plsc.load_gather(rb0, [a >> 3, a & seven])
                a = q10 + kk
                c10 = plsc.load_gather(rb1, [a >> 3, a & seven])
                a = q11 + kk
                c11 = plsc.load_gather(rb1, [a >> 3, a & seven])
                top = c00 + fu * (c01 - c00)
                bot = c10 + fu * (c11 - c10)
                res.append(top + fv * (bot - top))
            ox[sl] = res[0]
            oy[sl] = res[1]
            oz[sl] = res[2]
            return carry2

        lax.fori_loop(0, NG, blend_group, 0, unroll=4)

    def dirs_of(s):
        return (s["dx"], s["dy"], s["dz"])

    def outs_of(s):
        return (s["ox"], s["oy"], s["oz"])

    def fire_dirs(c, s):
        for cp in dir_copies(c, dirs_of(s), s["semd"]):
            cp.start()

    def calc_fire(c, s):
        for cp in dir_copies(c, dirs_of(s), s["semd"]):
            cp.wait()
        calc_chunk(dirs_of(s), s["ib0"], s["ib1"], s["pk"], s["fu"], s["fv"])
        for cp in gather_copies(s["ib0"], s["ib1"], s["rb0"], s["rb1"],
                                s["semg"]):
            cp.start()

    def drain_blend(c, s, first):
        @pl.when(jnp.logical_not(first))
        def _():
            for cp in out_copies(c - 4, outs_of(s), s["semo"]):
                cp.wait()
        for cp in gather_copies(s["ib0"], s["ib1"], s["rb0"], s["rb1"],
                                s["semg"]):
            cp.wait()
        blend_chunk(s["rb0"], s["rb1"], s["pk"], s["fu"], s["fv"], outs_of(s))
        for cp in out_copies(c, outs_of(s), s["semo"]):
            cp.start()

    A, B, Cs, D = sets
    fire_dirs(0, A)
    fire_dirs(1, B)
    fire_dirs(2, Cs)
    fire_dirs(3, D)
    calc_fire(0, A)
    calc_fire(1, B)

    def quad_body(j, carry):
        c = j * 4
        calc_fire(c + 2, Cs)
        calc_fire(c + 3, D)

        @pl.when(c + 4 < NCHUNK)
        def _():
            fire_dirs(c + 4, A)
            fire_dirs(c + 5, B)
        drain_blend(c, A, j == 0)
        drain_blend(c + 1, B, j == 0)

        @pl.when(c + 4 < NCHUNK)
        def _():
            calc_fire(c + 4, A)
            calc_fire(c + 5, B)

        @pl.when(c + 6 < NCHUNK)
        def _():
            fire_dirs(c + 6, Cs)
            fire_dirs(c + 7, D)
        drain_blend(c + 2, Cs, j == 0)
        drain_blend(c + 3, D, j == 0)
        return carry

    lax.fori_loop(0, NCHUNK // 4, quad_body, 0)
    for cp in out_copies(NCHUNK - 4, outs_of(A), A["semo"]):
        cp.wait()
    for cp in out_copies(NCHUNK - 3, outs_of(B), B["semo"]):
        cp.wait()
    for cp in out_copies(NCHUNK - 2, outs_of(Cs), Cs["semo"]):
        cp.wait()
    for cp in out_copies(NCHUNK - 1, outs_of(D), D["semo"]):
        cp.wait()


@jax.jit
def _run(lf, tbf):
    mesh = plsc.VectorSubcoreMesh(core_axis_name="c", subcore_axis_name="s")
    cp = pltpu.CompilerParams(needs_layout_passes=False,
                              use_tc_tiling_on_sc=False)
    fmt = pl.kernel(
        _fmt_body,
        out_type=jax.ShapeDtypeStruct((6 * R * R * 3,), jnp.float32),
        mesh=mesh,
        compiler_params=cp,
        scratch_types=(
            [pltpu.VMEM((TROW,), jnp.float32)] * 6
            + [pltpu.VMEM((3 * TROW,), jnp.float32)] * 2
            + [pltpu.SemaphoreType.DMA] * 4
        ),
    )
    tab = fmt(tbf).reshape(-1, 8)
    f = pl.kernel(
        _sc_body,
        out_type=jax.ShapeDtypeStruct((N * 3,), jnp.float32),
        mesh=mesh,
        compiler_params=cp,
        scratch_types=(
            [
                t for _ in range(4) for t in (
                    [pltpu.VMEM((CH,), jnp.float32)] * 3
                    + [pltpu.VMEM((2 * CH,), jnp.int32)] * 2
                    + [pltpu.VMEM((CH,), jnp.int32)]
                    + [pltpu.VMEM((CH,), jnp.float32)] * 2
                    + [pltpu.VMEM((2 * CH, 8), jnp.float32)] * 2
                    + [pltpu.VMEM((CH,), jnp.float32)] * 3
                    + [pltpu.SemaphoreType.DMA] * 3
                )
            ]
        ),
    )
    return f(lf, tab)


def kernel(l, base):
    lf = (l.transpose(2, 0, 1).reshape(3, 128, 8, 8, 128)
           .transpose(0, 1, 3, 2, 4).reshape(-1))
    tbf = (base.transpose(0, 3, 1, 2).reshape(6, 3, 128, 8, 8, 128)
               .transpose(0, 1, 2, 4, 3, 5).reshape(-1))
    out = _run(lf, tbf)
    return (out.reshape(3, 128, 8, 8, 128).transpose(0, 1, 3, 2, 4)
               .reshape(3, 1024, 1024).transpose(1, 2, 0))
